# baseline (speedup 1.0000x reference)
import functools

import jax
import jax.numpy as jnp
from jax import lax
from jax.experimental import pallas as pl
from jax.experimental.pallas import tpu as pltpu
from jax.experimental.pallas import tpu_sc as plsc

DIM = 768
BATCH = 16384
NUM_CORES = 2
NUM_SUBCORES = 16
NW = NUM_CORES * NUM_SUBCORES
B_PER_W = BATCH // NW
CHUNK = 64
N_CHUNKS = B_PER_W // CHUNK


@functools.partial(
    pl.kernel,
    out_type=jax.ShapeDtypeStruct((BATCH, DIM), jnp.float32),
    mesh=plsc.VectorSubcoreMesh(core_axis_name="c", subcore_axis_name="s"),
    scratch_types=[
        pltpu.VMEM((B_PER_W,), jnp.int32),
        pltpu.VMEM((2, CHUNK, DIM), jnp.float32),
        pltpu.SemaphoreType.DMA,
        pltpu.SemaphoreType.DMA,
        pltpu.SemaphoreType.DMA,
        pltpu.SemaphoreType.DMA,
    ],
)
def _sc_lookup(ids_hbm, table_hbm, out_hbm, idx_v, rows_v, g0, g1, s0, s1):
    wid = lax.axis_index("s") * NUM_CORES + lax.axis_index("c")
    base = wid * B_PER_W
    pltpu.sync_copy(ids_hbm.at[pl.ds(base, B_PER_W)], idx_v)

    gsems = [g0, g1]
    ssems = [s0, s1]
    gathers = [None, None]
    scatters = [None, None]
    for c in range(N_CHUNKS):
        buf = c % 2
        if scatters[buf] is not None:
            scatters[buf].wait()
        gathers[buf] = pltpu.async_copy(
            table_hbm.at[idx_v.at[pl.ds(c * CHUNK, CHUNK)]],
            rows_v.at[buf],
            gsems[buf],
        )
        gathers[buf].wait()
        scatters[buf] = pltpu.async_copy(
            rows_v.at[buf],
            out_hbm.at[pl.ds(base + c * CHUNK, CHUNK)],
            ssems[buf],
        )
    for buf in range(2):
        if scatters[buf] is not None:
            scatters[buf].wait()


def kernel(step_ids, step_embeddings):
    out = _sc_lookup(step_ids.astype(jnp.int32), step_embeddings)
    return out[:, None, :]


# speedup vs baseline: 4.5396x; 4.5396x over previous
import functools

import jax
import jax.numpy as jnp
from jax import lax
from jax.experimental import pallas as pl
from jax.experimental.pallas import tpu as pltpu
from jax.experimental.pallas import tpu_sc as plsc

DIM = 768
NUM_STEPS = 4
BATCH = 16384
NUM_CORES = 2
NUM_SUBCORES = 16
NW = NUM_CORES * NUM_SUBCORES
B_PER_W = BATCH // NW
CHUNK = 64
N_CHUNKS = B_PER_W // CHUNK
LANES = 16
N_GROUPS = B_PER_W // LANES
REP_PER_WORKER = 8
REP_PER_CORE = NUM_SUBCORES * REP_PER_WORKER
REP_TOTAL = NUM_CORES * REP_PER_CORE
REP_ROWS = REP_TOTAL * NUM_STEPS


@functools.partial(
    pl.kernel,
    out_type=(
        jax.ShapeDtypeStruct((BATCH, 1, DIM), jnp.float32),
        jax.ShapeDtypeStruct((REP_ROWS, 1, DIM), jnp.float32),
    ),
    mesh=plsc.VectorSubcoreMesh(core_axis_name="c", subcore_axis_name="s"),
    scratch_types=[
        pltpu.VMEM((B_PER_W,), jnp.int32),
        pltpu.VMEM((2, CHUNK, 1, DIM), jnp.float32),
        pltpu.VMEM((NUM_STEPS, 1, DIM), jnp.float32),
        pltpu.SemaphoreType.DMA,
        pltpu.SemaphoreType.DMA,
        pltpu.SemaphoreType.DMA,
        pltpu.SemaphoreType.DMA,
        pltpu.SemaphoreType.DMA,
    ],
)
def _sc_lookup(ids_hbm, table_hbm, out_hbm, rep_hbm,
               idx_v, rows_v, table_v, g0, g1, s0, s1, rsem):
    cid = lax.axis_index("c")
    sid = lax.axis_index("s")
    wid = sid * NUM_CORES + cid
    base = wid * B_PER_W

    pltpu.sync_copy(table_hbm, table_v)
    pltpu.sync_copy(ids_hbm.at[pl.ds(base, B_PER_W)], idx_v)

    rep0 = (cid * NUM_SUBCORES + sid) * REP_PER_WORKER
    rep_copies = []
    for k in range(REP_PER_WORKER):
        rep_copies.append(pltpu.async_copy(
            table_v,
            rep_hbm.at[pl.ds((rep0 + k) * NUM_STEPS, NUM_STEPS)],
            rsem,
        ))

    lane = lax.broadcasted_iota(jnp.int32, (LANES,), 0)
    core_off = (cid.astype(jnp.int32) * REP_PER_CORE) * NUM_STEPS
    for g in range(N_GROUPS):
        rep = ((g * LANES + lane) % REP_PER_CORE) * NUM_STEPS + core_off
        idx_v[pl.ds(g * LANES, LANES)] = idx_v[pl.ds(g * LANES, LANES)] + rep

    for cp in rep_copies:
        cp.wait()
    plsc.subcore_barrier()

    gsems = [g0, g1]
    ssems = [s0, s1]
    scatters = [None, None]
    for c in range(N_CHUNKS):
        buf = c % 2
        if scatters[buf] is not None:
            scatters[buf].wait()
        pltpu.async_copy(
            rep_hbm.at[idx_v.at[pl.ds(c * CHUNK, CHUNK)]],
            rows_v.at[buf],
            gsems[buf],
        ).wait()
        scatters[buf] = pltpu.async_copy(
            rows_v.at[buf],
            out_hbm.at[pl.ds(base + c * CHUNK, CHUNK)],
            ssems[buf],
        )
    for buf in range(2):
        if scatters[buf] is not None:
            scatters[buf].wait()


def kernel(step_ids, step_embeddings):
    out, _ = _sc_lookup(step_ids.astype(jnp.int32), step_embeddings[:, None, :])
    return out


# speedup vs baseline: 4.6419x; 1.0225x over previous
import functools

import jax
import jax.numpy as jnp
from jax import lax
from jax.experimental import pallas as pl
from jax.experimental.pallas import tpu as pltpu
from jax.experimental.pallas import tpu_sc as plsc

DIM = 768
NUM_STEPS = 4
BATCH = 16384
NUM_CORES = 2
NUM_SUBCORES = 16
NW = NUM_CORES * NUM_SUBCORES
B_PER_W = BATCH // NW
CHUNK = 64
N_CHUNKS = B_PER_W // CHUNK
LANES = 16
N_GROUPS = B_PER_W // LANES
REP_PER_WORKER = 8
REP_PER_CORE = NUM_SUBCORES * REP_PER_WORKER
REP_TOTAL = NUM_CORES * REP_PER_CORE
REP_ROWS = REP_TOTAL * NUM_STEPS


@functools.partial(
    pl.kernel,
    out_type=jax.ShapeDtypeStruct((BATCH, 1, DIM), jnp.float32),
    mesh=plsc.VectorSubcoreMesh(core_axis_name="c", subcore_axis_name="s"),
    scratch_types=[
        pltpu.HBM((REP_ROWS, 1, DIM), jnp.float32),
        pltpu.VMEM((B_PER_W,), jnp.int32),
        pltpu.VMEM((2, CHUNK, 1, DIM), jnp.float32),
        pltpu.VMEM((NUM_STEPS, 1, DIM), jnp.float32),
        pltpu.SemaphoreType.DMA,
        pltpu.SemaphoreType.DMA,
        pltpu.SemaphoreType.DMA,
        pltpu.SemaphoreType.DMA,
        pltpu.SemaphoreType.DMA,
    ],
)
def _sc_lookup(ids_hbm, table_hbm, out_hbm, rep_hbm,
               idx_v, rows_v, table_v, g0, g1, s0, s1, rsem):
    cid = lax.axis_index("c")
    sid = lax.axis_index("s")
    wid = sid * NUM_CORES + cid
    base = wid * B_PER_W

    pltpu.sync_copy(table_hbm, table_v)
    pltpu.sync_copy(ids_hbm.at[pl.ds(base, B_PER_W)], idx_v)

    rep0 = (cid * NUM_SUBCORES + sid) * REP_PER_WORKER
    rep_copies = []
    for k in range(REP_PER_WORKER):
        rep_copies.append(pltpu.async_copy(
            table_v,
            rep_hbm.at[pl.ds((rep0 + k) * NUM_STEPS, NUM_STEPS)],
            rsem,
        ))

    lane = lax.broadcasted_iota(jnp.int32, (LANES,), 0)
    core_off = (cid.astype(jnp.int32) * REP_PER_CORE) * NUM_STEPS
    for g in range(N_GROUPS):
        rep = ((g * LANES + lane) % REP_PER_CORE) * NUM_STEPS + core_off
        idx_v[pl.ds(g * LANES, LANES)] = idx_v[pl.ds(g * LANES, LANES)] + rep

    for cp in rep_copies:
        cp.wait()
    plsc.subcore_barrier()

    gsems = [g0, g1]
    ssems = [s0, s1]
    scatters = [None, None]
    gathers = [None, None]

    def start_gather(c):
        gathers[c % 2] = pltpu.async_copy(
            rep_hbm.at[idx_v.at[pl.ds(c * CHUNK, CHUNK)]],
            rows_v.at[c % 2],
            gsems[c % 2],
        )

    start_gather(0)
    for c in range(N_CHUNKS):
        buf = c % 2
        nxt = (c + 1) % 2
        if c + 1 < N_CHUNKS:
            if scatters[nxt] is not None:
                scatters[nxt].wait()
            start_gather(c + 1)
        gathers[buf].wait()
        scatters[buf] = pltpu.async_copy(
            rows_v.at[buf],
            out_hbm.at[pl.ds(base + c * CHUNK, CHUNK)],
            ssems[buf],
        )
    for buf in range(2):
        if scatters[buf] is not None:
            scatters[buf].wait()


def kernel(step_ids, step_embeddings):
    return _sc_lookup(step_ids.astype(jnp.int32), step_embeddings[:, None, :])


# speedup vs baseline: 4.8446x; 1.0437x over previous
import functools

import jax
import jax.numpy as jnp
from jax import lax
from jax.experimental import pallas as pl
from jax.experimental.pallas import tpu as pltpu
from jax.experimental.pallas import tpu_sc as plsc

DIM = 768
NUM_STEPS = 4
BATCH = 16384
NUM_CORES = 2
NUM_SUBCORES = 16
NW = NUM_CORES * NUM_SUBCORES
B_PER_W = BATCH // NW
CHUNK = 64
N_CHUNKS = B_PER_W // CHUNK
LANES = 16
N_GROUPS = B_PER_W // LANES
REP_PER_WORKER = 8
REP_TOTAL = NW * REP_PER_WORKER
REP_ROWS = REP_TOTAL * NUM_STEPS


@functools.partial(
    pl.kernel,
    out_type=jax.ShapeDtypeStruct((BATCH, 1, DIM), jnp.float32),
    mesh=plsc.VectorSubcoreMesh(core_axis_name="c", subcore_axis_name="s"),
    scratch_types=[
        pltpu.HBM((REP_ROWS, 1, DIM), jnp.float32),
        pltpu.VMEM((B_PER_W,), jnp.int32),
        pltpu.VMEM((2, CHUNK, 1, DIM), jnp.float32),
        pltpu.VMEM((NUM_STEPS, 1, DIM), jnp.float32),
        pltpu.SemaphoreType.DMA,
        pltpu.SemaphoreType.DMA,
        pltpu.SemaphoreType.DMA,
        pltpu.SemaphoreType.DMA,
        pltpu.SemaphoreType.DMA,
    ],
)
def _sc_lookup(ids_hbm, table_hbm, out_hbm, rep_hbm,
               idx_v, rows_v, table_v, g0, g1, s0, s1, rsem):
    cid = lax.axis_index("c")
    sid = lax.axis_index("s")
    wid = sid * NUM_CORES + cid
    base = wid * B_PER_W

    pltpu.sync_copy(table_hbm, table_v)
    rep0 = wid * REP_PER_WORKER
    rep_copies = []
    for k in range(REP_PER_WORKER):
        rep_copies.append(pltpu.async_copy(
            table_v,
            rep_hbm.at[pl.ds((rep0 + k) * NUM_STEPS, NUM_STEPS)],
            rsem,
        ))

    pltpu.sync_copy(ids_hbm.at[pl.ds(base, B_PER_W)], idx_v)
    lane = lax.broadcasted_iota(jnp.int32, (LANES,), 0)
    for g in range(N_GROUPS):
        rep = (rep0 + (g * LANES + lane) % REP_PER_WORKER) * NUM_STEPS
        idx_v[pl.ds(g * LANES, LANES)] = idx_v[pl.ds(g * LANES, LANES)] + rep

    for cp in rep_copies:
        cp.wait()

    gsems = [g0, g1]
    ssems = [s0, s1]
    scatters = [None, None]
    gathers = [None, None]

    def start_gather(c):
        gathers[c % 2] = pltpu.async_copy(
            rep_hbm.at[idx_v.at[pl.ds(c * CHUNK, CHUNK)]],
            rows_v.at[c % 2],
            gsems[c % 2],
        )

    start_gather(0)
    for c in range(N_CHUNKS):
        buf = c % 2
        nxt = (c + 1) % 2
        if c + 1 < N_CHUNKS:
            if scatters[nxt] is not None:
                scatters[nxt].wait()
            start_gather(c + 1)
        gathers[buf].wait()
        scatters[buf] = pltpu.async_copy(
            rows_v.at[buf],
            out_hbm.at[pl.ds(base + c * CHUNK, CHUNK)],
            ssems[buf],
        )
    for buf in range(2):
        if scatters[buf] is not None:
            scatters[buf].wait()


def kernel(step_ids, step_embeddings):
    return _sc_lookup(step_ids.astype(jnp.int32), step_embeddings[:, None, :])


# speedup vs baseline: 4.8794x; 1.0072x over previous
import functools

import jax
import jax.numpy as jnp
from jax import lax
from jax.experimental import pallas as pl
from jax.experimental.pallas import tpu as pltpu
from jax.experimental.pallas import tpu_sc as plsc

DIM = 768
NUM_STEPS = 4
BATCH = 16384
NUM_CORES = 2
NUM_SUBCORES = 16
NW = NUM_CORES * NUM_SUBCORES
B_PER_W = BATCH // NW
CHUNK = 32
N_CHUNKS = B_PER_W // CHUNK
NBUF = 4
LANES = 16
N_GROUPS = B_PER_W // LANES
REP_PER_WORKER = 8
REP_TOTAL = NW * REP_PER_WORKER
REP_ROWS = REP_TOTAL * NUM_STEPS


@functools.partial(
    pl.kernel,
    out_type=jax.ShapeDtypeStruct((BATCH, 1, DIM), jnp.float32),
    mesh=plsc.VectorSubcoreMesh(core_axis_name="c", subcore_axis_name="s"),
    scratch_types=[
        pltpu.HBM((REP_ROWS, 1, DIM), jnp.float32),
        pltpu.VMEM((B_PER_W,), jnp.int32),
        pltpu.VMEM((NBUF, CHUNK, 1, DIM), jnp.float32),
        pltpu.VMEM((NUM_STEPS, 1, DIM), jnp.float32),
        pltpu.SemaphoreType.DMA,
        pltpu.SemaphoreType.DMA,
        pltpu.SemaphoreType.DMA,
        pltpu.SemaphoreType.DMA,
        pltpu.SemaphoreType.DMA,
        pltpu.SemaphoreType.DMA,
        pltpu.SemaphoreType.DMA,
        pltpu.SemaphoreType.DMA,
        pltpu.SemaphoreType.DMA,
    ],
)
def _sc_lookup(ids_hbm, table_hbm, out_hbm, rep_hbm,
               idx_v, rows_v, table_v, g0, g1, g2, g3, s0, s1, s2, s3, rsem):
    cid = lax.axis_index("c")
    sid = lax.axis_index("s")
    wid = sid * NUM_CORES + cid
    base = wid * B_PER_W

    pltpu.sync_copy(table_hbm, table_v)
    rep0 = wid * REP_PER_WORKER
    rep_copies = []
    for k in range(REP_PER_WORKER):
        rep_copies.append(pltpu.async_copy(
            table_v,
            rep_hbm.at[pl.ds((rep0 + k) * NUM_STEPS, NUM_STEPS)],
            rsem,
        ))

    pltpu.sync_copy(ids_hbm.at[pl.ds(base, B_PER_W)], idx_v)
    lane = lax.broadcasted_iota(jnp.int32, (LANES,), 0)
    for g in range(N_GROUPS):
        rep = (rep0 + (g * LANES + lane) % REP_PER_WORKER) * NUM_STEPS
        idx_v[pl.ds(g * LANES, LANES)] = idx_v[pl.ds(g * LANES, LANES)] + rep

    for cp in rep_copies:
        cp.wait()

    gsems = [g0, g1, g2, g3]
    ssems = [s0, s1, s2, s3]
    scatters = [None] * NBUF
    gathers = [None] * NBUF

    def start_gather(c):
        b = c % NBUF
        if scatters[b] is not None:
            scatters[b].wait()
            scatters[b] = None
        gathers[b] = pltpu.async_copy(
            rep_hbm.at[idx_v.at[pl.ds(c * CHUNK, CHUNK)]],
            rows_v.at[b],
            gsems[b],
        )

    for c in range(NBUF - 1):
        start_gather(c)
    for c in range(N_CHUNKS):
        buf = c % NBUF
        if c + NBUF - 1 < N_CHUNKS:
            start_gather(c + NBUF - 1)
        gathers[buf].wait()
        scatters[buf] = pltpu.async_copy(
            rows_v.at[buf],
            out_hbm.at[pl.ds(base + c * CHUNK, CHUNK)],
            ssems[buf],
        )
    for buf in range(NBUF):
        if scatters[buf] is not None:
            scatters[buf].wait()


def kernel(step_ids, step_embeddings):
    return _sc_lookup(step_ids.astype(jnp.int32), step_embeddings[:, None, :])
